# Initial kernel scaffold; baseline (speedup 1.0000x reference)
import jax
import jax.numpy as jnp
from jax.experimental import pallas as pl


def kernel(y, edge_index, edge_weight, b0_W0, b0_W1, b0_W2, b0_bias, b0_gamma, b0_beta, b1_W0, b1_W1, b1_W2, b1_bias, b1_gamma, b1_beta):
    raise NotImplementedError("write your pallas kernel here")


# speedup vs baseline: 2.1536x; 2.1536x over previous
import functools
import jax
import jax.numpy as jnp
from jax import lax
from jax.experimental import pallas as pl
from jax.experimental.pallas import tpu as pltpu
from jax.experimental.pallas import tpu_sc as plsc

N = 10000
E = 160000
D = 256
DH = 128
NS = 16
C = 128
E_PAD = 163840
EPT = E_PAD // NS
NCHUNK = EPT // C
ROWS_PT = 624
RB = 1000


def _spmv_body(h_hbm, ei_hbm, ew_hbm, z_hbm, o_hbm,
               src_v, dst_v, ew_v, rows_v, acc_sh, sem):
    cid = lax.axis_index("c")
    sid = lax.axis_index("s")

    @pl.when(sid < NS - 1)
    def _():
        pltpu.sync_copy(z_hbm.at[pl.ds(0, ROWS_PT)],
                        acc_sh.at[pl.ds(sid * ROWS_PT, ROWS_PT)])

    @pl.when(sid == NS - 1)
    def _():
        pltpu.sync_copy(z_hbm, acc_sh.at[pl.ds((NS - 1) * ROWS_PT, 640)])

    plsc.subcore_barrier()

    e_base = sid * EPT

    def chunk(i, carry):
        e0 = e_base + i * C
        pltpu.sync_copy(ei_hbm.at[0, pl.ds(e0, C)], src_v)
        pltpu.sync_copy(ei_hbm.at[1, pl.ds(e0, C)], dst_v)
        pltpu.sync_copy(ew_hbm.at[pl.ds(e0, C)], ew_v)

        @pl.when(cid == 0)
        def _():
            pltpu.async_copy(h_hbm.at[0].at[src_v], rows_v, sem).wait()

        @pl.when(cid == 1)
        def _():
            pltpu.async_copy(h_hbm.at[1].at[src_v], rows_v, sem).wait()

        def scale_grp(g, c2):
            w16 = ew_v[pl.ds(g * 16, 16)]
            for j in range(16):
                w = w16[j]
                r = g * 16 + j
                for c8 in range(DH // 16):
                    sl = pl.ds(c8 * 16, 16)
                    rows_v[r, sl] = rows_v[r, sl] * w
            return c2

        lax.fori_loop(0, C // 16, scale_grp, 0)

        pltpu.sync_copy(rows_v, acc_sh.at[dst_v], add=True)
        return carry

    lax.fori_loop(0, NCHUNK, chunk, 0)
    plsc.subcore_barrier()

    last = NS - 1
    for c in range(2):
        @pl.when((cid == c) & (sid < last))
        def _(c=c):
            pltpu.sync_copy(acc_sh.at[pl.ds(sid * ROWS_PT, ROWS_PT)],
                            o_hbm.at[c, pl.ds(sid * ROWS_PT, ROWS_PT)])

        @pl.when((cid == c) & (sid == last))
        def _(c=c):
            pltpu.sync_copy(acc_sh.at[pl.ds(last * ROWS_PT, 640)],
                            o_hbm.at[c, pl.ds(last * ROWS_PT, 640)])


_spmv = functools.partial(
    pl.kernel,
    out_type=jax.ShapeDtypeStruct((2, N, DH), jnp.float32),
    mesh=plsc.VectorSubcoreMesh(core_axis_name="c", subcore_axis_name="s"),
    scratch_types=[
        pltpu.VMEM((C,), jnp.int32),
        pltpu.VMEM((C,), jnp.int32),
        pltpu.VMEM((C,), jnp.float32),
        pltpu.VMEM((C, DH), jnp.float32),
        pltpu.VMEM_SHARED((N, DH), jnp.float32),
        pltpu.SemaphoreType.DMA,
    ],
)(_spmv_body)


def _mm_body(x_ref, h1_ref, h2_ref, w0_ref, w1_ref, w2_ref, b_ref,
             s_ref, sums_ref):
    i = pl.program_id(0)
    s = jnp.dot(x_ref[0], w0_ref[:DH], preferred_element_type=jnp.float32)
    s += jnp.dot(x_ref[1], w0_ref[DH:], preferred_element_type=jnp.float32)
    s += jnp.dot(h1_ref[0], w1_ref[:DH], preferred_element_type=jnp.float32)
    s += jnp.dot(h1_ref[1], w1_ref[DH:], preferred_element_type=jnp.float32)
    s += jnp.dot(h2_ref[0], w2_ref[:DH], preferred_element_type=jnp.float32)
    s += jnp.dot(h2_ref[1], w2_ref[DH:], preferred_element_type=jnp.float32)
    s += b_ref[0:1]
    s_ref[...] = s

    @pl.when(i == 0)
    def _():
        sums_ref[...] = jnp.zeros_like(sums_ref)

    sums_ref[0:1] += jnp.sum(s, axis=0, keepdims=True)
    sums_ref[1:2] += jnp.sum(s * s, axis=0, keepdims=True)


_mm = pl.pallas_call(
    _mm_body,
    grid=(N // RB,),
    in_specs=[
        pl.BlockSpec((2, RB, DH), lambda i: (0, i, 0)),
        pl.BlockSpec((2, RB, DH), lambda i: (0, i, 0)),
        pl.BlockSpec((2, RB, DH), lambda i: (0, i, 0)),
        pl.BlockSpec((D, D), lambda i: (0, 0)),
        pl.BlockSpec((D, D), lambda i: (0, 0)),
        pl.BlockSpec((D, D), lambda i: (0, 0)),
        pl.BlockSpec((1, D), lambda i: (0, 0)),
    ],
    out_specs=[
        pl.BlockSpec((RB, D), lambda i: (i, 0)),
        pl.BlockSpec((2, D), lambda i: (0, 0)),
    ],
    out_shape=[
        jax.ShapeDtypeStruct((N, D), jnp.float32),
        jax.ShapeDtypeStruct((2, D), jnp.float32),
    ],
)


def _bn_body(x_ref, s_ref, sums_ref, g_ref, bt_ref, out_ref, osp_ref):
    mean = sums_ref[0:1] * (1.0 / N)
    var = sums_ref[1:2] * (1.0 / N) - mean * mean
    scale = g_ref[0:1] * lax.rsqrt(var + 1e-5)
    h = (s_ref[...] - mean) * scale + bt_ref[0:1]
    h = jnp.where(h >= 0, h, 0.01 * h)
    x_full = jnp.concatenate([x_ref[0], x_ref[1]], axis=1)
    o = x_full + h
    out_ref[...] = o
    osp_ref[0] = o[:, :DH]
    osp_ref[1] = o[:, DH:]


_bn = pl.pallas_call(
    _bn_body,
    grid=(N // RB,),
    in_specs=[
        pl.BlockSpec((2, RB, DH), lambda i: (0, i, 0)),
        pl.BlockSpec((RB, D), lambda i: (i, 0)),
        pl.BlockSpec((2, D), lambda i: (0, 0)),
        pl.BlockSpec((1, D), lambda i: (0, 0)),
        pl.BlockSpec((1, D), lambda i: (0, 0)),
    ],
    out_specs=[
        pl.BlockSpec((RB, D), lambda i: (i, 0)),
        pl.BlockSpec((2, RB, DH), lambda i: (0, i, 0)),
    ],
    out_shape=[
        jax.ShapeDtypeStruct((N, D), jnp.float32),
        jax.ShapeDtypeStruct((2, N, DH), jnp.float32),
    ],
)


def kernel(y, edge_index, edge_weight,
           b0_W0, b0_W1, b0_W2, b0_bias, b0_gamma, b0_beta,
           b1_W0, b1_W1, b1_W2, b1_bias, b1_gamma, b1_beta):
    ei = jnp.pad(edge_index, ((0, 0), (0, E_PAD - E)))
    ew = jnp.pad(edge_weight, (0, E_PAD - E))
    zeros = jnp.zeros((640, DH), jnp.float32)

    x = jnp.stack([y[:, :DH], y[:, DH:]])
    out = None
    for (W0, W1, W2, bias, gamma, beta) in (
        (b0_W0, b0_W1, b0_W2, b0_bias, b0_gamma, b0_beta),
        (b1_W0, b1_W1, b1_W2, b1_bias, b1_gamma, b1_beta),
    ):
        h1 = _spmv(x, ei, ew, zeros)
        h2 = _spmv(h1, ei, ew, zeros)
        s, sums = _mm(x, h1, h2, W0, W1, W2, bias[None, :])
        out, x = _bn(x, s, sums, gamma[None, :], beta[None, :])
    return out


# speedup vs baseline: 2.9319x; 1.3614x over previous
import functools
import jax
import jax.numpy as jnp
from jax import lax
from jax.experimental import pallas as pl
from jax.experimental.pallas import tpu as pltpu
from jax.experimental.pallas import tpu_sc as plsc

N = 10000
E = 160000
D = 256
DH = 128
NS = 16
C = 128
E_PAD = 163840
EPT = E_PAD // NS
NCHUNK = EPT // C
ROWS_PT = 624
RB = 1000


def _spmv_body(h_hbm, comb_hbm, ew_hbm, z_hbm, o_hbm,
               comb_v, ew_v, rows_v, acc_sh,
               gsem0, gsem1, ssem0, ssem1, isem0, isem1):
    cid = lax.axis_index("c")
    sid = lax.axis_index("s")
    gsem = (gsem0, gsem1)
    ssem = (ssem0, ssem1)
    isem = (isem0, isem1)

    @pl.when(sid < NS - 1)
    def _():
        pltpu.sync_copy(z_hbm.at[pl.ds(0, ROWS_PT)],
                        acc_sh.at[pl.ds(sid * ROWS_PT, ROWS_PT)])

    @pl.when(sid == NS - 1)
    def _():
        pltpu.sync_copy(z_hbm, acc_sh.at[pl.ds((NS - 1) * ROWS_PT, 640)])

    c_base = sid * NCHUNK

    def start_comb(i, b):
        pltpu.async_copy(comb_hbm.at[c_base + i], comb_v.at[b], isem[b])
        pltpu.async_copy(ew_hbm.at[c_base + i], ew_v.at[b], isem[b])

    def wait_comb(b):
        pltpu.make_async_copy(comb_hbm.at[0], comb_v.at[b], isem[b]).wait()
        pltpu.make_async_copy(ew_hbm.at[0], ew_v.at[b], isem[b]).wait()

    def start_gather(i, b):
        idx = comb_v.at[b, 0]

        @pl.when(cid == 0)
        def _():
            pltpu.async_copy(h_hbm.at[0].at[idx], rows_v.at[b], gsem[b])

        @pl.when(cid == 1)
        def _():
            pltpu.async_copy(h_hbm.at[1].at[idx], rows_v.at[b], gsem[b])

    def wait_gather(b):
        pltpu.make_async_copy(h_hbm.at[0].at[pl.ds(0, C)],
                              rows_v.at[b], gsem[b]).wait()

    def wait_scatter(b):
        pltpu.make_async_copy(h_hbm.at[0].at[pl.ds(0, C)],
                              rows_v.at[b], ssem[b]).wait()

    pltpu.sync_copy(comb_hbm.at[c_base], comb_v.at[0])
    pltpu.sync_copy(ew_hbm.at[c_base], ew_v.at[0])
    start_gather(0, 0)
    plsc.subcore_barrier()

    def pair(j, carry):
        for b in range(2):
            i = j * 2 + b
            nb = 1 - b

            @pl.when(i >= 1)
            def _():
                wait_scatter(nb)

            @pl.when(i + 1 < NCHUNK)
            def _():
                start_comb(i + 1, nb)

            wait_gather(b)

            def scale_grp(g, c2):
                w16 = ew_v[b, pl.ds(g * 16, 16)]
                for jj in range(16):
                    w = w16[jj]
                    r = g * 16 + jj
                    for c8 in range(DH // 16):
                        sl = pl.ds(c8 * 16, 16)
                        rows_v[b, r, sl] = rows_v[b, r, sl] * w
                return c2

            lax.fori_loop(0, C // 16, scale_grp, 0)

            pltpu.async_copy(rows_v.at[b], acc_sh.at[comb_v.at[b, 1]],
                             ssem[b], add=True)

            @pl.when(i + 1 < NCHUNK)
            def _():
                wait_comb(nb)
                start_gather(i + 1, nb)
        return carry

    lax.fori_loop(0, NCHUNK // 2, pair, 0)
    wait_scatter((NCHUNK - 1) % 2)
    plsc.subcore_barrier()

    last = NS - 1
    for c in range(2):
        @pl.when((cid == c) & (sid < last))
        def _(c=c):
            pltpu.sync_copy(acc_sh.at[pl.ds(sid * ROWS_PT, ROWS_PT)],
                            o_hbm.at[c, pl.ds(sid * ROWS_PT, ROWS_PT)])

        @pl.when((cid == c) & (sid == last))
        def _(c=c):
            pltpu.sync_copy(acc_sh.at[pl.ds(last * ROWS_PT, 640)],
                            o_hbm.at[c, pl.ds(last * ROWS_PT, 640)])


_spmv = functools.partial(
    pl.kernel,
    out_type=jax.ShapeDtypeStruct((2, N, DH), jnp.float32),
    mesh=plsc.VectorSubcoreMesh(core_axis_name="c", subcore_axis_name="s"),
    scratch_types=[
        pltpu.VMEM((2, 2, C), jnp.int32),
        pltpu.VMEM((2, C), jnp.float32),
        pltpu.VMEM((2, C, DH), jnp.float32),
        pltpu.VMEM_SHARED((N, DH), jnp.float32),
        pltpu.SemaphoreType.DMA,
        pltpu.SemaphoreType.DMA,
        pltpu.SemaphoreType.DMA,
        pltpu.SemaphoreType.DMA,
        pltpu.SemaphoreType.DMA,
        pltpu.SemaphoreType.DMA,
    ],
)(_spmv_body)


def _mm_body(x_ref, h1_ref, h2_ref, w0_ref, w1_ref, w2_ref, b_ref,
             s_ref, sums_ref):
    i = pl.program_id(0)
    s = jnp.dot(x_ref[0], w0_ref[:DH], preferred_element_type=jnp.float32)
    s += jnp.dot(x_ref[1], w0_ref[DH:], preferred_element_type=jnp.float32)
    s += jnp.dot(h1_ref[0], w1_ref[:DH], preferred_element_type=jnp.float32)
    s += jnp.dot(h1_ref[1], w1_ref[DH:], preferred_element_type=jnp.float32)
    s += jnp.dot(h2_ref[0], w2_ref[:DH], preferred_element_type=jnp.float32)
    s += jnp.dot(h2_ref[1], w2_ref[DH:], preferred_element_type=jnp.float32)
    s += b_ref[0:1]
    s_ref[...] = s

    @pl.when(i == 0)
    def _():
        sums_ref[...] = jnp.zeros_like(sums_ref)

    sums_ref[0:1] += jnp.sum(s, axis=0, keepdims=True)
    sums_ref[1:2] += jnp.sum(s * s, axis=0, keepdims=True)


_mm = pl.pallas_call(
    _mm_body,
    grid=(N // RB,),
    in_specs=[
        pl.BlockSpec((2, RB, DH), lambda i: (0, i, 0)),
        pl.BlockSpec((2, RB, DH), lambda i: (0, i, 0)),
        pl.BlockSpec((2, RB, DH), lambda i: (0, i, 0)),
        pl.BlockSpec((D, D), lambda i: (0, 0)),
        pl.BlockSpec((D, D), lambda i: (0, 0)),
        pl.BlockSpec((D, D), lambda i: (0, 0)),
        pl.BlockSpec((1, D), lambda i: (0, 0)),
    ],
    out_specs=[
        pl.BlockSpec((RB, D), lambda i: (i, 0)),
        pl.BlockSpec((2, D), lambda i: (0, 0)),
    ],
    out_shape=[
        jax.ShapeDtypeStruct((N, D), jnp.float32),
        jax.ShapeDtypeStruct((2, D), jnp.float32),
    ],
)


def _bn_body(x_ref, s_ref, sums_ref, g_ref, bt_ref, out_ref, osp_ref):
    mean = sums_ref[0:1] * (1.0 / N)
    var = sums_ref[1:2] * (1.0 / N) - mean * mean
    scale = g_ref[0:1] * lax.rsqrt(var + 1e-5)
    h = (s_ref[...] - mean) * scale + bt_ref[0:1]
    h = jnp.where(h >= 0, h, 0.01 * h)
    x_full = jnp.concatenate([x_ref[0], x_ref[1]], axis=1)
    o = x_full + h
    out_ref[...] = o
    osp_ref[0] = o[:, :DH]
    osp_ref[1] = o[:, DH:]


_bn = pl.pallas_call(
    _bn_body,
    grid=(N // RB,),
    in_specs=[
        pl.BlockSpec((2, RB, DH), lambda i: (0, i, 0)),
        pl.BlockSpec((RB, D), lambda i: (i, 0)),
        pl.BlockSpec((2, D), lambda i: (0, 0)),
        pl.BlockSpec((1, D), lambda i: (0, 0)),
        pl.BlockSpec((1, D), lambda i: (0, 0)),
    ],
    out_specs=[
        pl.BlockSpec((RB, D), lambda i: (i, 0)),
        pl.BlockSpec((2, RB, DH), lambda i: (0, i, 0)),
    ],
    out_shape=[
        jax.ShapeDtypeStruct((N, D), jnp.float32),
        jax.ShapeDtypeStruct((2, N, DH), jnp.float32),
    ],
)


def kernel(y, edge_index, edge_weight,
           b0_W0, b0_W1, b0_W2, b0_bias, b0_gamma, b0_beta,
           b1_W0, b1_W1, b1_W2, b1_bias, b1_gamma, b1_beta):
    src = jnp.pad(edge_index[0], (0, E_PAD - E)).reshape(E_PAD // C, C)
    dst = jnp.pad(edge_index[1], (0, E_PAD - E)).reshape(E_PAD // C, C)
    ew2 = jnp.pad(edge_weight, (0, E_PAD - E)).reshape(E_PAD // C, C)
    comb = jnp.stack([src, dst], axis=1)
    zeros = jnp.zeros((640, DH), jnp.float32)

    x = jnp.stack([y[:, :DH], y[:, DH:]])
    out = None
    for (W0, W1, W2, bias, gamma, beta) in (
        (b0_W0, b0_W1, b0_W2, b0_bias, b0_gamma, b0_beta),
        (b1_W0, b1_W1, b1_W2, b1_bias, b1_gamma, b1_beta),
    ):
        h1 = _spmv(x, comb, ew2, zeros)
        h2 = _spmv(h1, comb, ew2, zeros)
        s, sums = _mm(x, h1, h2, W0, W1, W2, bias[None, :])
        out, x = _bn(x, s, sums, gamma[None, :], beta[None, :])
    return out
